# baseline (speedup 1.0000x reference)
import math

import jax
import jax.numpy as jnp
from jax import lax
from jax.experimental import pallas as pl
from jax.experimental.pallas import tpu as pltpu
from jax.experimental.pallas import tpu_sc as plsc

N = 50000
D = 512
E = 800000
G = 512
OUT = 7
_BN_SCALE = 1.0 / math.sqrt(1.0 + 1e-05)

R = 1568
NB = 32
NP = R * NB
P2 = NP // 2
R2 = R // 2
CHW = 128
CH = 400
IB = 25
NBUF = 5
NSL = CH // IB
EPT = CH * CHW
EP = EPT * 16
RPT = NP // 16
RC = RPT // 8


def _pre_body(xa_ref, xb_ref, wa_ref, ba_ref, wb_ref, bb_ref, out_ref):
    i = pl.program_id(0)

    def mlp(xblk):
        t = jnp.maximum(
            jnp.dot(xblk, wa_ref[...], preferred_element_type=jnp.float32)
            + ba_ref[...], 0.0)
        return jnp.maximum(
            jnp.dot(t, wb_ref[...], preferred_element_type=jnp.float32)
            + bb_ref[...], 0.0)

    nxa = mlp(xa_ref[...])
    nxb = mlp(xb_ref[...])
    rows = lax.broadcasted_iota(jnp.int32, (R2, 64), 0) + i * R2
    nxb = jnp.where(rows < N - P2, nxb, 0.0)
    out_ref[...] = jnp.concatenate([nxa, nxb], axis=1)


def _pre_call(x, wa, ba, wb, bb):
    return pl.pallas_call(
        _pre_body,
        grid=(NB,),
        in_specs=[
            pl.BlockSpec((R2, D), lambda i: (i, 0)),
            pl.BlockSpec((R2, D), lambda i: (i + NB, 0)),
            pl.BlockSpec((D, 32), lambda i: (0, 0)),
            pl.BlockSpec((1, 32), lambda i: (0, 0)),
            pl.BlockSpec((32, 64), lambda i: (0, 0)),
            pl.BlockSpec((1, 64), lambda i: (0, 0)),
        ],
        out_specs=pl.BlockSpec((R2, 128), lambda i: (i, 0)),
        out_shape=jax.ShapeDtypeStruct((P2, 128), jnp.float32),
    )(x, x, wa, ba, wb, bb)


def _seg_body(table_hbm, srcidx_hbm, dstidx_hbm, zeros_hbm, out_hbm,
              src_v, dst_v, r0, r1, r2, r3, r4, acc_sh,
              g0, g1, g2, g3, g4, s0, s1, s2, s3, s4):
    c = lax.axis_index("c")
    s = lax.axis_index("s")
    w = c * 16 + s
    rows = (r0, r1, r2, r3, r4)
    gsem = (g0, g1, g2, g3, g4)
    ssem = (s0, s1, s2, s3, s4)
    ngrp = IB // NBUF

    def _gath(j, b, sem):
        return pltpu.make_async_copy(table_hbm.at[src_v.at[j]], rows[b], sem)

    def _scat(j, b, sem):
        return pltpu.make_async_copy(rows[b], acc_sh.at[dst_v.at[j]], sem)

    base = s * RPT
    pltpu.sync_copy(zeros_hbm, acc_sh.at[pl.ds(base, RPT)])
    plsc.subcore_barrier()

    def _slab(st, carry):
        pltpu.sync_copy(srcidx_hbm.at[w * NSL + st], src_v)
        pltpu.sync_copy(dstidx_hbm.at[s * NSL + st], dst_v)
        for b in range(3):
            _gath(b, b, gsem[b]).start()

        def _grp(g, carry2):
            for b in range(NBUF):
                j = g * NBUF + b
                bo = (b + 3) % NBUF
                _gath(j, b, gsem[b]).wait()
                pltpu.async_copy(rows[b], acc_sh.at[dst_v.at[j]], ssem[b],
                                 add=True)
                if b < 2:
                    @pl.when(g > 0)
                    def _():
                        _scat(j - 2, bo, ssem[bo]).wait()
                    _gath(j + 3, bo, gsem[bo]).start()
                else:
                    _scat(j - 2, bo, ssem[bo]).wait()

                    @pl.when(g < ngrp - 1)
                    def _():
                        _gath(j + 3, bo, gsem[bo]).start()
            return carry2

        lax.fori_loop(0, ngrp, _grp, 0)
        for b in (3, 4):
            _scat(0, b, ssem[b]).wait()
        return carry

    lax.fori_loop(0, NSL, _slab, 0)
    plsc.subcore_barrier()

    col = c + jnp.where(s >= 8, 2, 0)
    rb = base - jnp.where(s >= 8, P2, 0)
    pltpu.sync_copy(acc_sh.at[pl.ds(base, RPT)],
                    out_hbm.at[pl.ds(rb, RPT), col])


def _seg_call(tablep, srcidx, dstidx, zeros):
    table_flat = tablep.reshape(2 * NP, 32)
    mesh = plsc.VectorSubcoreMesh(core_axis_name="c", subcore_axis_name="s",
                                  num_cores=2, num_subcores=16)
    out = pl.kernel(
        _seg_body,
        out_type=jax.ShapeDtypeStruct((P2, 4, 32), jnp.float32),
        mesh=mesh,
        scratch_types=(
            [pltpu.VMEM((IB, CHW), jnp.int32),
             pltpu.VMEM((IB, CHW), jnp.int32)]
            + [pltpu.VMEM((CHW, 32), jnp.float32) for _ in range(NBUF)]
            + [pltpu.VMEM_SHARED((NP, 32), jnp.float32)]
            + [pltpu.SemaphoreType.DMA for _ in range(2 * NBUF)]
        ),
        compiler_params=pltpu.CompilerParams(use_tc_tiling_on_sc=False),
    )(table_flat, srcidx, dstidx, zeros)
    return out.reshape(P2, 128)


def _gin0_body(nx_ref, agg_ref, w1_ref, a1_ref, c1_ref, w2_ref, b2_ref,
               out_ref):
    nxp = nx_ref[...]
    h = nxp + agg_ref[...]
    t = jnp.maximum(jnp.dot(h, w1_ref[...], preferred_element_type=jnp.float32)
                    * a1_ref[...] + c1_ref[...], 0.0)
    out_ref[...] = (jnp.dot(t, w2_ref[...], preferred_element_type=jnp.float32)
                    + b2_ref[...] + nxp)


def _gin0_call(nx, agg, w1, a1, c1, w2, b2):
    specp = pl.BlockSpec((R2, 128), lambda i: (i, 0))
    return pl.pallas_call(
        _gin0_body,
        grid=(NB,),
        in_specs=[
            specp, specp,
            pl.BlockSpec((128, 128), lambda i: (0, 0)),
            pl.BlockSpec((1, 128), lambda i: (0, 0)),
            pl.BlockSpec((1, 128), lambda i: (0, 0)),
            pl.BlockSpec((128, 128), lambda i: (0, 0)),
            pl.BlockSpec((1, 128), lambda i: (0, 0)),
        ],
        out_specs=specp,
        out_shape=jax.ShapeDtypeStruct((P2, 128), jnp.float32),
    )(nx, agg, w1, a1, c1, w2, b2)


def _fin_body(nx_ref, g0_ref, agg_ref, be_ref, bo_ref, w1_ref, a1_ref,
              c1_ref, w2_ref, b2_ref, wp1_ref, bp1_ref, wp2_ref, bp2_ref,
              out_ref, acc_ref):
    i = pl.program_id(0)
    nxp = nx_ref[...]
    g0p = g0_ref[...]
    h = g0p + agg_ref[...]
    t = jnp.maximum(jnp.dot(h, w1_ref[...], preferred_element_type=jnp.float32)
                    * a1_ref[...] + c1_ref[...], 0.0)
    g1 = (jnp.dot(t, w2_ref[...], preferred_element_type=jnp.float32)
          + b2_ref[...] + g0p + nxp)

    ones_col = jnp.ones((R2, 1), jnp.float32)
    zpad = jnp.zeros((R2, 63), jnp.float32)
    iot = lax.broadcasted_iota(jnp.int32, (G, R2), 0)
    be = be_ref[0]
    bo = bo_ref[0]
    oh_e = (iot == be).astype(jnp.float32)
    oh_o = (iot == bo).astype(jnp.float32)
    gp_e = jnp.concatenate([g1[:, :64], ones_col, zpad], axis=1)
    gp_o = jnp.concatenate([g1[:, 64:], ones_col, zpad], axis=1)
    contrib = (jnp.dot(oh_e, gp_e, preferred_element_type=jnp.float32)
               + jnp.dot(oh_o, gp_o, preferred_element_type=jnp.float32))

    @pl.when(i == 0)
    def _():
        acc_ref[...] = jnp.zeros((G, 128), jnp.float32)

    acc_ref[...] += contrib

    @pl.when(i == NB - 1)
    def _():
        a = acc_ref[...]
        mean = a[:, :64] / jnp.maximum(a[:, 64:65], 1.0)
        t2 = jnp.maximum(
            jnp.dot(mean, wp1_ref[...], preferred_element_type=jnp.float32)
            + bp1_ref[...], 0.0)
        o = (jnp.dot(t2, wp2_ref[...], preferred_element_type=jnp.float32)
             + bp2_ref[...])
        m = jnp.max(o, axis=1, keepdims=True)
        lse = m + jnp.log(jnp.sum(jnp.exp(o - m), axis=1, keepdims=True))
        out_ref[...] = o - lse


def _fin_call(nx, g0, agg, be3, bo3, w1, a1, c1, w2, b2, wp1, bp1, wp2, bp2):
    specp = pl.BlockSpec((R2, 128), lambda i: (i, 0))
    bspec = pl.BlockSpec((1, 1, R2), lambda i: (i, 0, 0))
    wfull = lambda r, c: pl.BlockSpec((r, c), lambda i: (0, 0))
    return pl.pallas_call(
        _fin_body,
        grid=(NB,),
        in_specs=[
            specp, specp, specp, bspec, bspec,
            wfull(128, 128), wfull(1, 128), wfull(1, 128), wfull(128, 128),
            wfull(1, 128), wfull(64, 32), wfull(1, 32), wfull(32, OUT),
            wfull(1, OUT),
        ],
        out_specs=pl.BlockSpec((G, OUT), lambda i: (0, 0)),
        out_shape=jax.ShapeDtypeStruct((G, OUT), jnp.float32),
        scratch_shapes=[pltpu.VMEM((G, 128), jnp.float32)],
    )(nx, g0, agg, be3, bo3, w1, a1, c1, w2, b2, wp1, bp1, wp2, bp2)


def kernel(x, edge_index, batch,
           W_pre1, b_pre1, W_pre2, b_pre2, W_pre3, b_pre3, W_pre4, b_pre4,
           gin0_W1, gin0_b1, gin0_g, gin0_bb, gin0_W2, gin0_b2, bn0_g, bn0_b,
           gin1_W1, gin1_b1, gin1_g, gin1_bb, gin1_W2, gin1_b2, bn1_g, bn1_b,
           W_post1, b_post1, W_post2, b_post2):
    f32 = jnp.float32
    s = _BN_SCALE

    wa = jnp.zeros((D, 32), f32)
    wa = wa.at[:D - 2, :16].set(W_pre3).at[D - 2:, 16:].set(W_pre1)
    ba = jnp.concatenate([b_pre3, b_pre1]).reshape(1, 32)
    wb = jnp.zeros((32, 64), f32)
    wb = wb.at[:16, :32].set(W_pre4).at[16:, 32:].set(W_pre2)
    bb = jnp.concatenate([b_pre4, b_pre2]).reshape(1, 64)

    def blkdiag(m):
        z = jnp.zeros((64, 64), f32)
        return jnp.concatenate(
            [jnp.concatenate([m, z], axis=1),
             jnp.concatenate([z, m], axis=1)], axis=0)

    def fold(gW1, gb1, gg, gbb, gW2, gb2, bng, bnb):
        a1 = gg * s
        c1 = gb1 * gg * s + gbb
        sc2 = bng * s
        w2 = gW2 * sc2[None, :]
        b2 = gb2 * sc2 + bnb
        return (blkdiag(gW1), jnp.tile(a1, 2).reshape(1, 128),
                jnp.tile(c1, 2).reshape(1, 128), blkdiag(w2),
                jnp.tile(b2, 2).reshape(1, 128))

    w10, a10, c10, w20, b20 = fold(gin0_W1, gin0_b1, gin0_g, gin0_bb,
                                   gin0_W2, gin0_b2, bn0_g, bn0_b)
    w11, a11, c11, w21, b21 = fold(gin1_W1, gin1_b1, gin1_g, gin1_bb,
                                   gin1_W2, gin1_b2, bn1_g, bn1_b)

    src = edge_index[0]
    dst = edge_index[1]
    pad = EP - E
    src_p = jnp.concatenate([src, jnp.zeros((pad,), jnp.int32)])
    dst_p = jnp.concatenate([dst, jnp.full((pad,), N, jnp.int32)])
    srow = jnp.where(src_p < P2, 4 * src_p, 4 * src_p - 4 * P2 + 2)
    src2 = srow.reshape(16, NSL, IB, CHW)
    srcidx = jnp.concatenate([src2, src2 + 1], axis=0).reshape(
        32 * NSL, IB, CHW)
    dstidx = dst_p.reshape(16 * NSL, IB, CHW)

    bpad = jnp.concatenate(
        [batch.astype(jnp.int32), jnp.full((NP - N,), G, jnp.int32)])
    be3 = bpad[:P2].reshape(NB, 1, R2)
    bo3 = bpad[P2:].reshape(NB, 1, R2)

    zeros = jnp.zeros((RPT, 32), jnp.float32)
    nx = _pre_call(x, wa, ba, wb, bb)
    agg0 = _seg_call(nx, srcidx, dstidx, zeros)
    g0 = _gin0_call(nx, agg0, w10, a10, c10, w20, b20)
    agg1 = _seg_call(g0, srcidx, dstidx, zeros)
    return _fin_call(nx, g0, agg1, be3, bo3, w11, a11, c11, w21, b21,
                     W_post1, b_post1.reshape(1, 32), W_post2,
                     b_post2.reshape(1, OUT))


# speedup vs baseline: 1.4349x; 1.4349x over previous
import math

import jax
import jax.numpy as jnp
from jax import lax
from jax.experimental import pallas as pl
from jax.experimental.pallas import tpu as pltpu
from jax.experimental.pallas import tpu_sc as plsc

N = 50000
D = 512
E = 800000
G = 512
OUT = 7
_BN_SCALE = 1.0 / math.sqrt(1.0 + 1e-05)

NB = 32
NP = 50176
Q = NP // 4
R4 = Q // NB
CHW = 128
CH = 400
IB = 25
NBUF = 5
NSL = CH // IB
EPT = CH * CHW
EP = EPT * 16
RPT = NP // 16


def _pre_body(xa_ref, xb_ref, xc_ref, xd_ref, wa_ref, ba_ref, wb_ref, bb_ref,
              out_ref):
    i = pl.program_id(0)

    def mlp(xblk):
        t = jnp.maximum(
            jnp.dot(xblk, wa_ref[...], preferred_element_type=jnp.float32)
            + ba_ref[...], 0.0)
        return jnp.maximum(
            jnp.dot(t, wb_ref[...], preferred_element_type=jnp.float32)
            + bb_ref[...], 0.0)

    nxa = mlp(xa_ref[...])
    nxb = mlp(xb_ref[...])
    nxc = mlp(xc_ref[...])
    nxd = mlp(xd_ref[...])
    rows = lax.broadcasted_iota(jnp.int32, (R4, 64), 0) + i * R4
    nxd = jnp.where(rows < N - 3 * Q, nxd, 0.0)
    out_ref[0] = jnp.concatenate(
        [nxa[:, :32], nxb[:, :32], nxc[:, :32], nxd[:, :32]], axis=1)
    out_ref[1] = jnp.concatenate(
        [nxa[:, 32:], nxb[:, 32:], nxc[:, 32:], nxd[:, 32:]], axis=1)


def _pre_call(x, wa, ba, wb, bb):
    return pl.pallas_call(
        _pre_body,
        grid=(NB,),
        in_specs=[
            pl.BlockSpec((R4, D), lambda i: (i, 0)),
            pl.BlockSpec((R4, D), lambda i: (i + NB, 0)),
            pl.BlockSpec((R4, D), lambda i: (i + 2 * NB, 0)),
            pl.BlockSpec((R4, D), lambda i: (i + 3 * NB, 0)),
            pl.BlockSpec((D, 32), lambda i: (0, 0)),
            pl.BlockSpec((1, 32), lambda i: (0, 0)),
            pl.BlockSpec((32, 64), lambda i: (0, 0)),
            pl.BlockSpec((1, 64), lambda i: (0, 0)),
        ],
        out_specs=pl.BlockSpec((2, R4, 128), lambda i: (0, i, 0)),
        out_shape=jax.ShapeDtypeStruct((2, Q, 128), jnp.float32),
    )(x, x, x, x, wa, ba, wb, bb)


def _seg_body(table_hbm, srcidx_hbm, dstidx_hbm, zeros_hbm, out_hbm,
              src_v, dst_v, r0, r1, r2, r3, r4, acc_sh,
              g0, g1, g2, g3, g4, s0, s1, s2, s3, s4):
    c = lax.axis_index("c")
    s = lax.axis_index("s")
    w = c * 16 + s
    rows = (r0, r1, r2, r3, r4)
    gsem = (g0, g1, g2, g3, g4)
    ssem = (s0, s1, s2, s3, s4)
    ngrp = IB // NBUF

    def _gath(j, b, sem):
        return pltpu.make_async_copy(table_hbm.at[src_v.at[j]], rows[b], sem)

    def _scat(j, b, sem):
        return pltpu.make_async_copy(rows[b], acc_sh.at[dst_v.at[j]], sem)

    base = s * RPT
    pltpu.sync_copy(zeros_hbm, acc_sh.at[pl.ds(base, RPT)])
    plsc.subcore_barrier()

    def _slab(st, carry):
        pltpu.sync_copy(srcidx_hbm.at[w * NSL + st], src_v)
        pltpu.sync_copy(dstidx_hbm.at[s * NSL + st], dst_v)
        for b in range(3):
            _gath(b, b, gsem[b]).start()

        def _grp(g, carry2):
            for b in range(NBUF):
                j = g * NBUF + b
                bo = (b + 3) % NBUF
                _gath(j, b, gsem[b]).wait()
                pltpu.async_copy(rows[b], acc_sh.at[dst_v.at[j]], ssem[b],
                                 add=True)
                if b < 2:
                    @pl.when(g > 0)
                    def _():
                        _scat(j - 2, bo, ssem[bo]).wait()
                    _gath(j + 3, bo, gsem[bo]).start()
                else:
                    _scat(j - 2, bo, ssem[bo]).wait()

                    @pl.when(g < ngrp - 1)
                    def _():
                        _gath(j + 3, bo, gsem[bo]).start()
            return carry2

        lax.fori_loop(0, ngrp, _grp, 0)
        for b in (3, 4):
            _scat(0, b, ssem[b]).wait()
        return carry

    lax.fori_loop(0, NSL, _slab, 0)
    plsc.subcore_barrier()

    pltpu.sync_copy(acc_sh.at[pl.ds(base, RPT)],
                    out_hbm.at[pl.ds(c * NP + base, RPT)])


def _seg_call(table3, srcidx, dstidx, zeros):
    table_flat = table3.reshape(2 * NP, 32)
    mesh = plsc.VectorSubcoreMesh(core_axis_name="c", subcore_axis_name="s",
                                  num_cores=2, num_subcores=16)
    out = pl.kernel(
        _seg_body,
        out_type=jax.ShapeDtypeStruct((2 * NP, 32), jnp.float32),
        mesh=mesh,
        scratch_types=(
            [pltpu.VMEM((IB, CHW), jnp.int32),
             pltpu.VMEM((IB, CHW), jnp.int32)]
            + [pltpu.VMEM((CHW, 32), jnp.float32) for _ in range(NBUF)]
            + [pltpu.VMEM_SHARED((NP, 32), jnp.float32)]
            + [pltpu.SemaphoreType.DMA for _ in range(2 * NBUF)]
        ),
        compiler_params=pltpu.CompilerParams(use_tc_tiling_on_sc=False),
    )(table_flat, srcidx, dstidx, zeros)
    return out.reshape(2, Q, 128)


def _gin_core(lob, hib, alo, ahi, w1lo_ref, w1hi_ref, a4_ref, c4_ref,
              w2lo_ref, w2hi_ref, b2lo_ref, b2hi_ref):
    hlo = lob + alo
    hhi = hib + ahi
    t4 = jnp.maximum(
        (jnp.dot(hlo, w1lo_ref[...], preferred_element_type=jnp.float32)
         + jnp.dot(hhi, w1hi_ref[...], preferred_element_type=jnp.float32))
        * a4_ref[...] + c4_ref[...], 0.0)
    glo = (jnp.dot(t4, w2lo_ref[...], preferred_element_type=jnp.float32)
           + b2lo_ref[...] + lob)
    ghi = (jnp.dot(t4, w2hi_ref[...], preferred_element_type=jnp.float32)
           + b2hi_ref[...] + hib)
    return glo, ghi


def _gin0_body(nx_ref, agg_ref, w1lo_ref, w1hi_ref, a4_ref, c4_ref,
               w2lo_ref, w2hi_ref, b2lo_ref, b2hi_ref, out_ref):
    glo, ghi = _gin_core(nx_ref[0], nx_ref[1], agg_ref[0], agg_ref[1],
                         w1lo_ref, w1hi_ref, a4_ref, c4_ref,
                         w2lo_ref, w2hi_ref, b2lo_ref, b2hi_ref)
    out_ref[0] = glo
    out_ref[1] = ghi


def _w_specs():
    wf = lambda r, c: pl.BlockSpec((r, c), lambda i: (0, 0))
    return [wf(128, 256), wf(128, 256), wf(1, 256), wf(1, 256),
            wf(256, 128), wf(256, 128), wf(1, 128), wf(1, 128)]


def _gin0_call(nx3, agg3, wpack):
    spec3 = pl.BlockSpec((2, R4, 128), lambda i: (0, i, 0))
    return pl.pallas_call(
        _gin0_body,
        grid=(NB,),
        in_specs=[spec3, spec3] + _w_specs(),
        out_specs=spec3,
        out_shape=jax.ShapeDtypeStruct((2, Q, 128), jnp.float32),
    )(nx3, agg3, *wpack)


def _fin_body(nx_ref, g0_ref, agg_ref, b0_ref, b1_ref, b2_ref, b3_ref,
              w1lo_ref, w1hi_ref, a4_ref, c4_ref, w2lo_ref, w2hi_ref,
              b2lo_ref, b2hi_ref, wp1_ref, bp1_ref, wp2_ref, bp2_ref,
              out_ref, acc_ref):
    i = pl.program_id(0)
    glo, ghi = _gin_core(g0_ref[0], g0_ref[1], agg_ref[0], agg_ref[1],
                         w1lo_ref, w1hi_ref, a4_ref, c4_ref,
                         w2lo_ref, w2hi_ref, b2lo_ref, b2hi_ref)
    g1lo = glo + nx_ref[0]
    g1hi = ghi + nx_ref[1]

    ones_col = jnp.ones((R4, 1), jnp.float32)
    zpad = jnp.zeros((R4, 63), jnp.float32)
    iot = lax.broadcasted_iota(jnp.int32, (G, R4), 0)
    contrib = jnp.zeros((G, 128), jnp.float32)
    for q, b_ref in enumerate((b0_ref, b1_ref, b2_ref, b3_ref)):
        oh = (iot == b_ref[0]).astype(jnp.float32)
        gq = jnp.concatenate(
            [g1lo[:, 32 * q:32 * q + 32], g1hi[:, 32 * q:32 * q + 32],
             ones_col, zpad], axis=1)
        contrib = contrib + jnp.dot(oh, gq,
                                    preferred_element_type=jnp.float32)

    @pl.when(i == 0)
    def _():
        acc_ref[...] = jnp.zeros((G, 128), jnp.float32)

    acc_ref[...] += contrib

    @pl.when(i == NB - 1)
    def _():
        a = acc_ref[...]
        mean = a[:, :64] / jnp.maximum(a[:, 64:65], 1.0)
        t2 = jnp.maximum(
            jnp.dot(mean, wp1_ref[...], preferred_element_type=jnp.float32)
            + bp1_ref[...], 0.0)
        o = (jnp.dot(t2, wp2_ref[...], preferred_element_type=jnp.float32)
             + bp2_ref[...])
        m = jnp.max(o, axis=1, keepdims=True)
        lse = m + jnp.log(jnp.sum(jnp.exp(o - m), axis=1, keepdims=True))
        out_ref[...] = o - lse


def _fin_call(nx3, g03, agg3, bqs, wpack, wp1, bp1, wp2, bp2):
    spec3 = pl.BlockSpec((2, R4, 128), lambda i: (0, i, 0))
    bspec = pl.BlockSpec((1, 1, R4), lambda i: (i, 0, 0))
    wf = lambda r, c: pl.BlockSpec((r, c), lambda i: (0, 0))
    return pl.pallas_call(
        _fin_body,
        grid=(NB,),
        in_specs=([spec3, spec3, spec3] + [bspec] * 4 + _w_specs()
                  + [wf(64, 32), wf(1, 32), wf(32, OUT), wf(1, OUT)]),
        out_specs=pl.BlockSpec((G, OUT), lambda i: (0, 0)),
        out_shape=jax.ShapeDtypeStruct((G, OUT), jnp.float32),
        scratch_shapes=[pltpu.VMEM((G, 128), jnp.float32)],
    )(nx3, g03, agg3, *bqs, *wpack, wp1, bp1, wp2, bp2)


def kernel(x, edge_index, batch,
           W_pre1, b_pre1, W_pre2, b_pre2, W_pre3, b_pre3, W_pre4, b_pre4,
           gin0_W1, gin0_b1, gin0_g, gin0_bb, gin0_W2, gin0_b2, bn0_g, bn0_b,
           gin1_W1, gin1_b1, gin1_g, gin1_bb, gin1_W2, gin1_b2, bn1_g, bn1_b,
           W_post1, b_post1, W_post2, b_post2):
    f32 = jnp.float32
    s = _BN_SCALE

    wa = jnp.zeros((D, 32), f32)
    wa = wa.at[:D - 2, :16].set(W_pre3).at[D - 2:, 16:].set(W_pre1)
    ba = jnp.concatenate([b_pre3, b_pre1]).reshape(1, 32)
    wb = jnp.zeros((32, 64), f32)
    wb = wb.at[:16, :32].set(W_pre4).at[16:, 32:].set(W_pre2)
    bb = jnp.concatenate([b_pre4, b_pre2]).reshape(1, 64)

    def blkdiag4(m):
        return jax.scipy.linalg.block_diag(m, m, m, m)

    def fold(gW1, gb1, gg, gbb, gW2, gb2, bng, bnb):
        a1 = gg * s
        c1 = gb1 * gg * s + gbb
        sc2 = bng * s
        w2 = gW2 * sc2[None, :]
        b2 = gb2 * sc2 + bnb
        return (blkdiag4(gW1[:32, :]), blkdiag4(gW1[32:, :]),
                jnp.tile(a1, 4).reshape(1, 256),
                jnp.tile(c1, 4).reshape(1, 256),
                blkdiag4(w2[:, :32]), blkdiag4(w2[:, 32:]),
                jnp.tile(b2[:32], 4).reshape(1, 128),
                jnp.tile(b2[32:], 4).reshape(1, 128))

    wpack0 = fold(gin0_W1, gin0_b1, gin0_g, gin0_bb, gin0_W2, gin0_b2,
                  bn0_g, bn0_b)
    wpack1 = fold(gin1_W1, gin1_b1, gin1_g, gin1_bb, gin1_W2, gin1_b2,
                  bn1_g, bn1_b)

    src = edge_index[0]
    dst = edge_index[1]
    pad = EP - E
    src_p = jnp.concatenate([src, jnp.zeros((pad,), jnp.int32)])
    dst_p = jnp.concatenate([dst, jnp.full((pad,), N, jnp.int32)])
    srow = 4 * (src_p % Q) + src_p // Q
    drow = 4 * (dst_p % Q) + dst_p // Q
    src2 = srow.reshape(16, NSL, IB, CHW)
    srcidx = jnp.concatenate([src2, src2 + NP], axis=0).reshape(
        32 * NSL, IB, CHW)
    dstidx = drow.reshape(16 * NSL, IB, CHW)

    bpad = jnp.concatenate(
        [batch.astype(jnp.int32), jnp.full((NP - N,), G, jnp.int32)])
    bqs = [bpad[qi * Q:(qi + 1) * Q].reshape(NB, 1, R4) for qi in range(4)]

    zeros = jnp.zeros((RPT, 32), f32)
    nx3 = _pre_call(x, wa, ba, wb, bb)
    agg0 = _seg_call(nx3, srcidx, dstidx, zeros)
    g03 = _gin0_call(nx3, agg0, wpack0)
    agg1 = _seg_call(g03, srcidx, dstidx, zeros)
    return _fin_call(nx3, g03, agg1, bqs, wpack1,
                     W_post1, b_post1.reshape(1, 32), W_post2,
                     b_post2.reshape(1, OUT))


# speedup vs baseline: 1.6625x; 1.1586x over previous
import math

import jax
import jax.numpy as jnp
from jax import lax
from jax.experimental import pallas as pl
from jax.experimental.pallas import tpu as pltpu
from jax.experimental.pallas import tpu_sc as plsc

N = 50000
D = 512
E = 800000
G = 512
OUT = 7
_BN_SCALE = 1.0 / math.sqrt(1.0 + 1e-05)

NB = 32
NP = 50176
Q = NP // 4
R4 = Q // NB
CHW = 256
CH = 198
IB = 9
NBUF = 3
NSL = CH // IB
EPT = CH * CHW
EP = EPT * 16
RPT = NP // 16


def _pre_body(xa_ref, xb_ref, xc_ref, xd_ref, wa_ref, ba_ref, wb_ref, bb_ref,
              out_ref):
    i = pl.program_id(0)

    def mlp(xblk):
        t = jnp.maximum(
            jnp.dot(xblk, wa_ref[...], preferred_element_type=jnp.float32)
            + ba_ref[...], 0.0)
        return jnp.maximum(
            jnp.dot(t, wb_ref[...], preferred_element_type=jnp.float32)
            + bb_ref[...], 0.0)

    nxa = mlp(xa_ref[...])
    nxb = mlp(xb_ref[...])
    nxc = mlp(xc_ref[...])
    nxd = mlp(xd_ref[...])
    rows = lax.broadcasted_iota(jnp.int32, (R4, 64), 0) + i * R4
    nxd = jnp.where(rows < N - 3 * Q, nxd, 0.0)
    out_ref[0] = jnp.concatenate(
        [nxa[:, :32], nxb[:, :32], nxc[:, :32], nxd[:, :32]], axis=1)
    out_ref[1] = jnp.concatenate(
        [nxa[:, 32:], nxb[:, 32:], nxc[:, 32:], nxd[:, 32:]], axis=1)


def _pre_call(x, wa, ba, wb, bb):
    return pl.pallas_call(
        _pre_body,
        grid=(NB,),
        in_specs=[
            pl.BlockSpec((R4, D), lambda i: (i, 0)),
            pl.BlockSpec((R4, D), lambda i: (i + NB, 0)),
            pl.BlockSpec((R4, D), lambda i: (i + 2 * NB, 0)),
            pl.BlockSpec((R4, D), lambda i: (i + 3 * NB, 0)),
            pl.BlockSpec((D, 32), lambda i: (0, 0)),
            pl.BlockSpec((1, 32), lambda i: (0, 0)),
            pl.BlockSpec((32, 64), lambda i: (0, 0)),
            pl.BlockSpec((1, 64), lambda i: (0, 0)),
        ],
        out_specs=pl.BlockSpec((2, R4, 128), lambda i: (0, i, 0)),
        out_shape=jax.ShapeDtypeStruct((2, Q, 128), jnp.float32),
    )(x, x, x, x, wa, ba, wb, bb)


def _seg_body(table_hbm, srcidx_hbm, dstidx_hbm, zeros_hbm, out_hbm,
              src_v, dst_v, r0, r1, r2, acc_sh,
              g0, g1, g2, s0, s1, s2):
    c = lax.axis_index("c")
    s = lax.axis_index("s")
    w = c * 16 + s
    rows = (r0, r1, r2)
    gsem = (g0, g1, g2)
    ssem = (s0, s1, s2)
    ngrp = IB // NBUF

    def _gath(j, b, sem):
        return pltpu.make_async_copy(table_hbm.at[src_v.at[j]], rows[b], sem)

    def _scat(j, b, sem):
        return pltpu.make_async_copy(rows[b], acc_sh.at[dst_v.at[j]], sem)

    base = s * RPT
    pltpu.sync_copy(zeros_hbm, acc_sh.at[pl.ds(base, RPT)])
    plsc.subcore_barrier()

    def _slab(st, carry):
        pltpu.sync_copy(srcidx_hbm.at[w * NSL + st], src_v)
        pltpu.sync_copy(dstidx_hbm.at[s * NSL + st], dst_v)
        for b in range(2):
            _gath(b, b, gsem[b]).start()

        def _grp(g, carry2):
            for b in range(NBUF):
                j = g * NBUF + b
                bo = (b + 2) % NBUF
                _gath(j, b, gsem[b]).wait()
                pltpu.async_copy(rows[b], acc_sh.at[dst_v.at[j]], ssem[b],
                                 add=True)
                if b == 0:
                    @pl.when(g > 0)
                    def _():
                        _scat(j - 1, bo, ssem[bo]).wait()
                    _gath(j + 2, bo, gsem[bo]).start()
                else:
                    _scat(j - 1, bo, ssem[bo]).wait()

                    @pl.when(g < ngrp - 1)
                    def _():
                        _gath(j + 2, bo, gsem[bo]).start()
            return carry2

        lax.fori_loop(0, ngrp, _grp, 0)
        _scat(0, (IB - 1) % NBUF, ssem[(IB - 1) % NBUF]).wait()
        return carry

    lax.fori_loop(0, NSL, _slab, 0)
    plsc.subcore_barrier()

    pltpu.sync_copy(acc_sh.at[pl.ds(base, RPT)],
                    out_hbm.at[pl.ds(c * NP + base, RPT)])


def _seg_call(table3, srcidx, dstidx, zeros):
    table_flat = table3.reshape(2 * NP, 32)
    mesh = plsc.VectorSubcoreMesh(core_axis_name="c", subcore_axis_name="s",
                                  num_cores=2, num_subcores=16)
    out = pl.kernel(
        _seg_body,
        out_type=jax.ShapeDtypeStruct((2 * NP, 32), jnp.float32),
        mesh=mesh,
        scratch_types=(
            [pltpu.VMEM((IB, CHW), jnp.int32),
             pltpu.VMEM((IB, CHW), jnp.int32)]
            + [pltpu.VMEM((CHW, 32), jnp.float32) for _ in range(NBUF)]
            + [pltpu.VMEM_SHARED((NP, 32), jnp.float32)]
            + [pltpu.SemaphoreType.DMA for _ in range(2 * NBUF)]
        ),
        name="segsum_sc",
        compiler_params=pltpu.CompilerParams(use_tc_tiling_on_sc=False),
    )(table_flat, srcidx, dstidx, zeros)
    return out.reshape(2, Q, 128)


def _gin_core(lob, hib, alo, ahi, w1lo_ref, w1hi_ref, a4_ref, c4_ref,
              w2lo_ref, w2hi_ref, b2lo_ref, b2hi_ref):
    hlo = lob + alo
    hhi = hib + ahi
    t4 = jnp.maximum(
        (jnp.dot(hlo, w1lo_ref[...], preferred_element_type=jnp.float32)
         + jnp.dot(hhi, w1hi_ref[...], preferred_element_type=jnp.float32))
        * a4_ref[...] + c4_ref[...], 0.0)
    glo = (jnp.dot(t4, w2lo_ref[...], preferred_element_type=jnp.float32)
           + b2lo_ref[...] + lob)
    ghi = (jnp.dot(t4, w2hi_ref[...], preferred_element_type=jnp.float32)
           + b2hi_ref[...] + hib)
    return glo, ghi


def _gin0_body(nx_ref, agg_ref, w1lo_ref, w1hi_ref, a4_ref, c4_ref,
               w2lo_ref, w2hi_ref, b2lo_ref, b2hi_ref, out_ref):
    glo, ghi = _gin_core(nx_ref[0], nx_ref[1], agg_ref[0], agg_ref[1],
                         w1lo_ref, w1hi_ref, a4_ref, c4_ref,
                         w2lo_ref, w2hi_ref, b2lo_ref, b2hi_ref)
    out_ref[0] = glo
    out_ref[1] = ghi


def _w_specs():
    wf = lambda r, c: pl.BlockSpec((r, c), lambda i: (0, 0))
    return [wf(128, 256), wf(128, 256), wf(1, 256), wf(1, 256),
            wf(256, 128), wf(256, 128), wf(1, 128), wf(1, 128)]


def _gin0_call(nx3, agg3, wpack):
    spec3 = pl.BlockSpec((2, R4, 128), lambda i: (0, i, 0))
    return pl.pallas_call(
        _gin0_body,
        grid=(NB,),
        in_specs=[spec3, spec3] + _w_specs(),
        out_specs=spec3,
        out_shape=jax.ShapeDtypeStruct((2, Q, 128), jnp.float32),
    )(nx3, agg3, *wpack)


def _fin_body(nx_ref, g0_ref, agg_ref, b0_ref, b1_ref, b2_ref, b3_ref,
              w1lo_ref, w1hi_ref, a4_ref, c4_ref, w2lo_ref, w2hi_ref,
              b2lo_ref, b2hi_ref, wp1_ref, bp1_ref, wp2_ref, bp2_ref,
              out_ref, acc_ref):
    i = pl.program_id(0)
    glo, ghi = _gin_core(g0_ref[0], g0_ref[1], agg_ref[0], agg_ref[1],
                         w1lo_ref, w1hi_ref, a4_ref, c4_ref,
                         w2lo_ref, w2hi_ref, b2lo_ref, b2hi_ref)
    g1lo = glo + nx_ref[0]
    g1hi = ghi + nx_ref[1]

    ones_col = jnp.ones((R4, 1), jnp.float32)
    zpad = jnp.zeros((R4, 63), jnp.float32)
    iot = lax.broadcasted_iota(jnp.int32, (G, R4), 0)
    contrib = jnp.zeros((G, 128), jnp.float32)
    for q, b_ref in enumerate((b0_ref, b1_ref, b2_ref, b3_ref)):
        oh = (iot == b_ref[0]).astype(jnp.float32)
        gq = jnp.concatenate(
            [g1lo[:, 32 * q:32 * q + 32], g1hi[:, 32 * q:32 * q + 32],
             ones_col, zpad], axis=1)
        contrib = contrib + jnp.dot(oh, gq,
                                    preferred_element_type=jnp.float32)

    @pl.when(i == 0)
    def _():
        acc_ref[...] = jnp.zeros((G, 128), jnp.float32)

    acc_ref[...] += contrib

    @pl.when(i == NB - 1)
    def _():
        a = acc_ref[...]
        mean = a[:, :64] / jnp.maximum(a[:, 64:65], 1.0)
        t2 = jnp.maximum(
            jnp.dot(mean, wp1_ref[...], preferred_element_type=jnp.float32)
            + bp1_ref[...], 0.0)
        o = (jnp.dot(t2, wp2_ref[...], preferred_element_type=jnp.float32)
             + bp2_ref[...])
        m = jnp.max(o, axis=1, keepdims=True)
        lse = m + jnp.log(jnp.sum(jnp.exp(o - m), axis=1, keepdims=True))
        out_ref[...] = o - lse


def _fin_call(nx3, g03, agg3, bqs, wpack, wp1, bp1, wp2, bp2):
    spec3 = pl.BlockSpec((2, R4, 128), lambda i: (0, i, 0))
    bspec = pl.BlockSpec((1, 1, R4), lambda i: (i, 0, 0))
    wf = lambda r, c: pl.BlockSpec((r, c), lambda i: (0, 0))
    return pl.pallas_call(
        _fin_body,
        grid=(NB,),
        in_specs=([spec3, spec3, spec3] + [bspec] * 4 + _w_specs()
                  + [wf(64, 32), wf(1, 32), wf(32, OUT), wf(1, OUT)]),
        out_specs=pl.BlockSpec((G, OUT), lambda i: (0, 0)),
        out_shape=jax.ShapeDtypeStruct((G, OUT), jnp.float32),
        scratch_shapes=[pltpu.VMEM((G, 128), jnp.float32)],
    )(nx3, g03, agg3, *bqs, *wpack, wp1, bp1, wp2, bp2)


def kernel(x, edge_index, batch,
           W_pre1, b_pre1, W_pre2, b_pre2, W_pre3, b_pre3, W_pre4, b_pre4,
           gin0_W1, gin0_b1, gin0_g, gin0_bb, gin0_W2, gin0_b2, bn0_g, bn0_b,
           gin1_W1, gin1_b1, gin1_g, gin1_bb, gin1_W2, gin1_b2, bn1_g, bn1_b,
           W_post1, b_post1, W_post2, b_post2):
    f32 = jnp.float32
    s = _BN_SCALE

    wa = jnp.zeros((D, 32), f32)
    wa = wa.at[:D - 2, :16].set(W_pre3).at[D - 2:, 16:].set(W_pre1)
    ba = jnp.concatenate([b_pre3, b_pre1]).reshape(1, 32)
    wb = jnp.zeros((32, 64), f32)
    wb = wb.at[:16, :32].set(W_pre4).at[16:, 32:].set(W_pre2)
    bb = jnp.concatenate([b_pre4, b_pre2]).reshape(1, 64)

    def blkdiag4(m):
        return jax.scipy.linalg.block_diag(m, m, m, m)

    def fold(gW1, gb1, gg, gbb, gW2, gb2, bng, bnb):
        a1 = gg * s
        c1 = gb1 * gg * s + gbb
        sc2 = bng * s
        w2 = gW2 * sc2[None, :]
        b2 = gb2 * sc2 + bnb
        return (blkdiag4(gW1[:32, :]), blkdiag4(gW1[32:, :]),
                jnp.tile(a1, 4).reshape(1, 256),
                jnp.tile(c1, 4).reshape(1, 256),
                blkdiag4(w2[:, :32]), blkdiag4(w2[:, 32:]),
                jnp.tile(b2[:32], 4).reshape(1, 128),
                jnp.tile(b2[32:], 4).reshape(1, 128))

    wpack0 = fold(gin0_W1, gin0_b1, gin0_g, gin0_bb, gin0_W2, gin0_b2,
                  bn0_g, bn0_b)
    wpack1 = fold(gin1_W1, gin1_b1, gin1_g, gin1_bb, gin1_W2, gin1_b2,
                  bn1_g, bn1_b)

    src = edge_index[0]
    dst = edge_index[1]
    pad = EP - E
    src_p = jnp.concatenate([src, jnp.zeros((pad,), jnp.int32)])
    dst_p = jnp.concatenate([dst, jnp.full((pad,), N, jnp.int32)])
    srow = 4 * (src_p % Q) + src_p // Q
    drow = 4 * (dst_p % Q) + dst_p // Q
    src2 = srow.reshape(16, NSL, IB, CHW)
    srcidx = jnp.concatenate([src2, src2 + NP], axis=0).reshape(
        32 * NSL, IB, CHW)
    dstidx = drow.reshape(16 * NSL, IB, CHW)

    bpad = jnp.concatenate(
        [batch.astype(jnp.int32), jnp.full((NP - N,), G, jnp.int32)])
    bqs = [bpad[qi * Q:(qi + 1) * Q].reshape(NB, 1, R4) for qi in range(4)]

    zeros = jnp.zeros((RPT, 32), f32)
    nx3 = _pre_call(x, wa, ba, wb, bb)
    agg0 = _seg_call(nx3, srcidx, dstidx, zeros)
    g03 = _gin0_call(nx3, agg0, wpack0)
    agg1 = _seg_call(g03, srcidx, dstidx, zeros)
    return _fin_call(nx3, g03, agg1, bqs, wpack1,
                     W_post1, b_post1.reshape(1, 32), W_post2,
                     b_post2.reshape(1, OUT))
